# Initial kernel scaffold; baseline (speedup 1.0000x reference)
import jax
import jax.numpy as jnp
from jax.experimental import pallas as pl


def kernel(x):
    raise NotImplementedError("write your pallas kernel here")


# speedup vs baseline: 2.1574x; 2.1574x over previous
import functools

import jax
import jax.numpy as jnp
from jax import lax
from jax.experimental import pallas as pl
from jax.experimental.pallas import tpu as pltpu
from jax.experimental.pallas import tpu_sc as plsc

_BINS = (-1.8, -1.5429, -1.2857, -1.0286, -0.7714, -0.5143, -0.2571, 0.0,
         0.2571, 0.5143, 0.7714, 1.0286, 1.2857, 1.5429, 1.8)
_NG = 16
_N = 4096
_L = 16
_NCHUNK = _N // _L
_PER_W = _N // _NG


@jax.jit
def _sc_dispatch(y):
    mesh = plsc.VectorSubcoreMesh(
        core_axis_name="c", subcore_axis_name="s", num_cores=1, num_subcores=16
    )

    @functools.partial(
        pl.kernel,
        out_type=(
            jax.ShapeDtypeStruct((_N,), jnp.int32),
            jax.ShapeDtypeStruct((_N,), jnp.int32),
            jax.ShapeDtypeStruct((_NG,), jnp.int32),
        ),
        mesh=mesh,
        compiler_params=pltpu.CompilerParams(needs_layout_passes=False),
        scratch_types=[
            pltpu.VMEM((_N,), jnp.float32),
            pltpu.VMEM((_N,), jnp.int32),
            pltpu.VMEM((_N + _L,), jnp.int32),
            pltpu.VMEM((_N,), jnp.int32),
            pltpu.VMEM((_L,), jnp.int32),
            pltpu.VMEM((_NG * _L,), jnp.int32),
            pltpu.VMEM((_PER_W,), jnp.int32),
            pltpu.VMEM_SHARED((_N + _NG,), jnp.int32),
            pltpu.VMEM_SHARED((_NG * _L,), jnp.int32),
        ],
    )
    def k(y_hbm, m_hbm, order_hbm, counts_hbm,
          y_loc, m_all, comp, tgt, tmp_row, cnt2d, bounce, order_sh, cnt_sh):
        g = lax.axis_index("s")
        pltpu.sync_copy(y_hbm, y_loc)

        iota = lax.iota(jnp.int32, _L)

        def scan_body(kk, carry):
            c, ovec = carry
            yv = y_loc[pl.ds(kk * _L, _L)]
            m = jnp.where(yv > _BINS[0], 1, 0)
            for b in _BINS[1:]:
                m = m + jnp.where(yv > b, 1, 0)
            m_all[pl.ds(kk * _L, _L)] = m
            eq = jnp.where(m == g, 1, 0)
            pc = plsc.cumsum(eq)
            pos = jnp.where(m == g, c + pc - 1, _N + iota)
            plsc.store_scatter(comp, [pos], iota + kk * _L)
            c = c + jnp.sum(eq)
            ovec = ovec + jnp.where(m < g, 1, 0)
            return c, ovec

        c, ovec = lax.fori_loop(
            0, _NCHUNK, scan_body,
            (jnp.int32(0), jnp.zeros((_L,), jnp.int32)))
        off = jnp.sum(ovec)

        base = pl.multiple_of(g * _PER_W, _PER_W)
        pltpu.sync_copy(m_all.at[pl.ds(base, _PER_W)],
                        m_hbm.at[pl.ds(base, _PER_W)])

        def tgt_body(kk, _):
            j = iota + kk * _L
            tgt[pl.ds(kk * _L, _L)] = jnp.where(j < c, off + j, _N + g)
            return 0

        lax.fori_loop(0, _NCHUNK, tgt_body, 0)
        pltpu.sync_copy(comp.at[pl.ds(0, _N)], order_sh.at[tgt])

        tmp_row[...] = jnp.zeros((_L,), jnp.int32) + c
        gbase = pl.multiple_of(g * _L, _L)
        pltpu.sync_copy(tmp_row, cnt_sh.at[pl.ds(gbase, _L)])

        plsc.subcore_barrier()

        pltpu.sync_copy(order_sh.at[pl.ds(base, _PER_W)], bounce)
        pltpu.sync_copy(bounce, order_hbm.at[pl.ds(base, _PER_W)])

        @pl.when(g == 0)
        def _():
            pltpu.sync_copy(cnt_sh, cnt2d)
            counts = jnp.zeros((_L,), jnp.int32)
            for l in range(_NG):
                counts = jnp.where(iota == l, cnt2d[pl.ds(l * _L, _L)], counts)
            tmp_row[...] = counts
            pltpu.sync_copy(tmp_row, counts_hbm)

    return k(y)


def kernel(x):
    y = x[0, :, 0]
    memberships, order, counts = _sc_dispatch(y)
    return memberships, order, counts
